# baseline (speedup 1.0000x reference)
import functools

import jax
import jax.numpy as jnp
from jax import lax
from jax.experimental import pallas as pl
from jax.experimental.pallas import tpu as pltpu
from jax.experimental.pallas import tpu_sc as plsc

BATCH = 4096
SEQ = 200
D = 64
CHUNK = 100
N_CHUNKS = BATCH * SEQ // CHUNK
NW = 32
PER_W = N_CHUNKS // NW
LANES = 16

_mesh = plsc.VectorSubcoreMesh(core_axis_name="c", subcore_axis_name="s")


@functools.partial(
    pl.kernel,
    mesh=_mesh,
    out_type=jax.ShapeDtypeStruct((N_CHUNKS, CHUNK, D), jnp.float32),
    scratch_types=[
        pltpu.VMEM((CHUNK,), jnp.int32),
        pltpu.VMEM((CHUNK, D), jnp.float32),
        pltpu.VMEM((2, CHUNK, D), jnp.float32),
        pltpu.SemaphoreType.DMA,
    ],
    compiler_params=pltpu.CompilerParams(use_tc_tiling_on_sc=False),
)
def _emb_kernel(idx_hbm, pos_hbm, table_hbm, out_hbm, idx_v, rows_v, pos_v, sem):
    wid = lax.axis_index("s") * 2 + lax.axis_index("c")
    base = wid * PER_W
    pltpu.sync_copy(pos_hbm, pos_v)

    def pair_body(c2, _):
        for p in range(2):
            chunk = base + c2 * 2 + p
            pltpu.sync_copy(idx_hbm.at[chunk], idx_v)
            pltpu.async_copy(table_hbm.at[idx_v], rows_v, sem).wait()

            def add_row(i, _):
                for j in range(D // LANES):
                    sl = pl.ds(j * LANES, LANES)
                    rows_v[i, sl] = rows_v[i, sl] + pos_v[p, i, sl]
                return 0

            lax.fori_loop(0, CHUNK, add_row, 0)
            pltpu.sync_copy(rows_v, out_hbm.at[chunk])
        return 0

    lax.fori_loop(0, PER_W // 2, pair_body, 0)


def kernel(x, token_emb_table, pos_emb):
    idx = x.reshape(N_CHUNKS, CHUNK)
    pos = pos_emb.reshape(SEQ // CHUNK, CHUNK, D)
    out = _emb_kernel(idx, pos, token_emb_table)
    return out.reshape(BATCH, SEQ, D)


# speedup vs baseline: 1.3059x; 1.3059x over previous
import functools

import jax
import jax.numpy as jnp
from jax import lax
from jax.experimental import pallas as pl
from jax.experimental.pallas import tpu as pltpu
from jax.experimental.pallas import tpu_sc as plsc

BATCH = 4096
SEQ = 200
D = 64
CHUNK = 100
N_CHUNKS = BATCH * SEQ // CHUNK
NW = 32
PER_W = N_CHUNKS // NW
LANES = 16
NBUF = 4
ROWS_PER_STEP = 4

_mesh = plsc.VectorSubcoreMesh(core_axis_name="c", subcore_axis_name="s")


@functools.partial(
    pl.kernel,
    mesh=_mesh,
    out_type=jax.ShapeDtypeStruct((N_CHUNKS, CHUNK, D), jnp.float32),
    scratch_types=[
        pltpu.VMEM((PER_W, CHUNK), jnp.int32),
        pltpu.VMEM((NBUF, CHUNK, D), jnp.float32),
        pltpu.VMEM((2, CHUNK, D), jnp.float32),
        [pltpu.SemaphoreType.DMA] * NBUF,
        [pltpu.SemaphoreType.DMA] * NBUF,
    ],
    compiler_params=pltpu.CompilerParams(use_tc_tiling_on_sc=False),
)
def _emb_kernel(idx_hbm, pos_hbm, table_hbm, out_hbm,
                idx_v, rows_v, pos_v, gsems, ssems):
    wid = lax.axis_index("s") * 2 + lax.axis_index("c")
    base = wid * PER_W
    pltpu.sync_copy(pos_hbm, pos_v)
    pltpu.sync_copy(idx_hbm.at[pl.ds(base, PER_W)], idx_v)

    def start_gather(k, b):
        pltpu.async_copy(table_hbm.at[idx_v.at[k]], rows_v.at[b], gsems[b])

    def wait_gather(k, b):
        pltpu.make_async_copy(
            table_hbm.at[idx_v.at[k]], rows_v.at[b], gsems[b]).wait()

    def start_store(k, b):
        pltpu.async_copy(rows_v.at[b], out_hbm.at[base + k], ssems[b])

    def wait_store(k, b):
        pltpu.make_async_copy(
            rows_v.at[b], out_hbm.at[base + k], ssems[b]).wait()

    def add_pos(b):
        p = b % 2

        def add_rows(i, _):
            for r in range(ROWS_PER_STEP):
                for j in range(D // LANES):
                    sl = pl.ds(j * LANES, LANES)
                    row = i * ROWS_PER_STEP + r
                    rows_v[b, row, sl] = rows_v[b, row, sl] + pos_v[p, row, sl]
            return 0

        lax.fori_loop(0, CHUNK // ROWS_PER_STEP, add_rows, 0)

    def stage(k, b, issue_j):
        wait_gather(k, b)
        add_pos(b)
        start_store(k, b)
        if issue_j:
            j = k + NBUF - 1
            bj = (b + NBUF - 1) % NBUF
            wait_store(j - NBUF, bj)
            start_gather(j, bj)

    for b in range(NBUF - 1):
        start_gather(b, b)

    wait_gather(0, 0)
    add_pos(0)
    start_store(0, 0)
    start_gather(NBUF - 1, NBUF - 1)
    for b in range(1, NBUF):
        stage(b, b, issue_j=True)

    def outer(k4, _):
        k0 = k4 * NBUF
        for b in range(NBUF):
            stage(k0 + b, b, issue_j=True)
        return 0

    lax.fori_loop(1, PER_W // NBUF - 1, outer, 0)

    kl = PER_W - NBUF
    stage(kl, 0, issue_j=True)
    for b in range(1, NBUF):
        stage(kl + b, b, issue_j=False)

    for b in range(NBUF):
        wait_store(kl + b, b)


def kernel(x, token_emb_table, pos_emb):
    idx = x.reshape(N_CHUNKS, CHUNK)
    pos = pos_emb.reshape(SEQ // CHUNK, CHUNK, D)
    out = _emb_kernel(idx, pos, token_emb_table)
    return out.reshape(BATCH, SEQ, D)


# speedup vs baseline: 1.3169x; 1.0084x over previous
import functools

import jax
import jax.numpy as jnp
from jax import lax
from jax.experimental import pallas as pl
from jax.experimental.pallas import tpu as pltpu
from jax.experimental.pallas import tpu_sc as plsc

BATCH = 4096
SEQ = 200
D = 64
NW = 32
PER_W = BATCH // NW
LANES = 16
NBUF = 4
ROWS_PER_STEP = 4

_mesh = plsc.VectorSubcoreMesh(core_axis_name="c", subcore_axis_name="s")


@functools.partial(
    pl.kernel,
    mesh=_mesh,
    out_type=jax.ShapeDtypeStruct((BATCH, SEQ, D), jnp.float32),
    scratch_types=[
        pltpu.VMEM((PER_W, 128), jnp.int32),
        pltpu.VMEM((PER_W, 72), jnp.int32),
        pltpu.VMEM((NBUF, SEQ, D), jnp.float32),
        pltpu.VMEM((1, SEQ, D), jnp.float32),
        [pltpu.SemaphoreType.DMA] * NBUF,
        [pltpu.SemaphoreType.DMA] * NBUF,
    ],
    compiler_params=pltpu.CompilerParams(use_tc_tiling_on_sc=False),
)
def _emb_kernel(idx_hbm, pos_hbm, table_hbm, out_hbm,
                idx_lo, idx_hi, rows_v, pos_v, gsems, ssems):
    wid = lax.axis_index("s") * 2 + lax.axis_index("c")
    base = wid * PER_W
    HL, HH = 128, 72
    pltpu.sync_copy(pos_hbm, pos_v)
    pltpu.sync_copy(idx_hbm.at[pl.ds(base, PER_W), pl.ds(0, HL)], idx_lo)
    pltpu.sync_copy(idx_hbm.at[pl.ds(base, PER_W), pl.ds(HL, HH)], idx_hi)

    def start_gather(k, b):
        pltpu.async_copy(
            table_hbm.at[idx_lo.at[k]], rows_v.at[b, pl.ds(0, HL)], gsems[b])
        pltpu.async_copy(
            table_hbm.at[idx_hi.at[k]], rows_v.at[b, pl.ds(HL, HH)], gsems[b])

    def wait_gather(k, b):
        pltpu.make_async_copy(
            table_hbm.at[idx_lo.at[k]], rows_v.at[b, pl.ds(0, HL)],
            gsems[b]).wait()
        pltpu.make_async_copy(
            table_hbm.at[idx_hi.at[k]], rows_v.at[b, pl.ds(HL, HH)],
            gsems[b]).wait()

    def start_store(k, b):
        pltpu.async_copy(rows_v.at[b], out_hbm.at[base + k], ssems[b])

    def wait_store(k, b):
        pltpu.make_async_copy(
            rows_v.at[b], out_hbm.at[base + k], ssems[b]).wait()

    def add_pos(b):
        def add_rows(i, _):
            for r in range(ROWS_PER_STEP):
                for j in range(D // LANES):
                    sl = pl.ds(j * LANES, LANES)
                    row = i * ROWS_PER_STEP + r
                    rows_v[b, row, sl] = rows_v[b, row, sl] + pos_v[0, row, sl]
            return 0

        lax.fori_loop(0, SEQ // ROWS_PER_STEP, add_rows, 0)

    def stage(k, b, issue_j):
        wait_gather(k, b)
        add_pos(b)
        start_store(k, b)
        if issue_j:
            j = k + NBUF - 1
            bj = (b + NBUF - 1) % NBUF
            wait_store(j - NBUF, bj)
            start_gather(j, bj)

    for b in range(NBUF - 1):
        start_gather(b, b)

    wait_gather(0, 0)
    add_pos(0)
    start_store(0, 0)
    start_gather(NBUF - 1, NBUF - 1)
    for b in range(1, NBUF):
        stage(b, b, issue_j=True)

    def outer(k4, _):
        k0 = k4 * NBUF
        for b in range(NBUF):
            stage(k0 + b, b, issue_j=True)
        return 0

    lax.fori_loop(1, PER_W // NBUF - 1, outer, 0)

    kl = PER_W - NBUF
    stage(kl, 0, issue_j=True)
    for b in range(1, NBUF):
        stage(kl + b, b, issue_j=False)

    for b in range(NBUF):
        wait_store(kl + b, b)


def kernel(x, token_emb_table, pos_emb):
    return _emb_kernel(x, pos_emb, token_emb_table)


# speedup vs baseline: 1.3210x; 1.0032x over previous
import functools

import jax
import jax.numpy as jnp
from jax import lax
from jax.experimental import pallas as pl
from jax.experimental.pallas import tpu as pltpu
from jax.experimental.pallas import tpu_sc as plsc

BATCH = 4096
SEQ = 200
D = 64
NW = 32
PER_W = BATCH // NW
LANES = 16
NBUF = 4
ROWS_PER_STEP = 4

_mesh = plsc.VectorSubcoreMesh(core_axis_name="c", subcore_axis_name="s")


@functools.partial(
    pl.kernel,
    mesh=_mesh,
    out_type=jax.ShapeDtypeStruct((BATCH, SEQ, D), jnp.float32),
    scratch_types=[
        pltpu.VMEM((PER_W, 128), jnp.int32),
        pltpu.VMEM((PER_W, 72), jnp.int32),
        pltpu.VMEM((NBUF, SEQ, D), jnp.float32),
        pltpu.VMEM((1, SEQ, D), jnp.float32),
        [pltpu.SemaphoreType.DMA] * NBUF,
        [pltpu.SemaphoreType.DMA] * NBUF,
    ],
    compiler_params=pltpu.CompilerParams(
        use_tc_tiling_on_sc=False, skip_device_barrier=True),
)
def _emb_kernel(idx_hbm, pos_hbm, table_hbm, out_hbm,
                idx_lo, idx_hi, rows_v, pos_v, gsems, ssems):
    wid = lax.axis_index("s") * 2 + lax.axis_index("c")
    base = wid * PER_W
    HL, HH = 128, 72
    pltpu.sync_copy(pos_hbm, pos_v)
    pltpu.sync_copy(idx_hbm.at[pl.ds(base, PER_W), pl.ds(0, HL)], idx_lo)
    pltpu.sync_copy(idx_hbm.at[pl.ds(base, PER_W), pl.ds(HL, HH)], idx_hi)

    def start_gather(k, b):
        pltpu.async_copy(
            table_hbm.at[idx_lo.at[k]], rows_v.at[b, pl.ds(0, HL)], gsems[b])
        pltpu.async_copy(
            table_hbm.at[idx_hi.at[k]], rows_v.at[b, pl.ds(HL, HH)], gsems[b])

    def wait_gather(k, b):
        pltpu.make_async_copy(
            table_hbm.at[idx_lo.at[k]], rows_v.at[b, pl.ds(0, HL)],
            gsems[b]).wait()
        pltpu.make_async_copy(
            table_hbm.at[idx_hi.at[k]], rows_v.at[b, pl.ds(HL, HH)],
            gsems[b]).wait()

    def start_store(k, b):
        pltpu.async_copy(rows_v.at[b], out_hbm.at[base + k], ssems[b])

    def wait_store(k, b):
        pltpu.make_async_copy(
            rows_v.at[b], out_hbm.at[base + k], ssems[b]).wait()

    def add_pos(b):
        def add_rows(i, _):
            for r in range(ROWS_PER_STEP):
                for j in range(D // LANES):
                    sl = pl.ds(j * LANES, LANES)
                    row = i * ROWS_PER_STEP + r
                    rows_v[b, row, sl] = rows_v[b, row, sl] + pos_v[0, row, sl]
            return 0

        lax.fori_loop(0, SEQ // ROWS_PER_STEP, add_rows, 0)

    def stage(k, b, issue_j):
        wait_gather(k, b)
        add_pos(b)
        start_store(k, b)
        if issue_j:
            j = k + NBUF - 1
            bj = (b + NBUF - 1) % NBUF
            wait_store(j - NBUF, bj)
            start_gather(j, bj)

    for b in range(NBUF - 1):
        start_gather(b, b)

    wait_gather(0, 0)
    add_pos(0)
    start_store(0, 0)
    start_gather(NBUF - 1, NBUF - 1)
    for b in range(1, NBUF):
        stage(b, b, issue_j=True)

    def outer(k4, _):
        k0 = k4 * NBUF
        for b in range(NBUF):
            stage(k0 + b, b, issue_j=True)
        return 0

    lax.fori_loop(1, PER_W // NBUF - 1, outer, 0)

    kl = PER_W - NBUF
    stage(kl, 0, issue_j=True)
    for b in range(1, NBUF):
        stage(kl + b, b, issue_j=False)

    for b in range(NBUF):
        wait_store(kl + b, b)


def kernel(x, token_emb_table, pos_emb):
    return _emb_kernel(x, pos_emb, token_emb_table)
